# baseline (speedup 1.0000x reference)
import functools

import jax
import jax.numpy as jnp
from jax import lax
from jax.experimental import pallas as pl
from jax.experimental.pallas import tpu as pltpu
from jax.experimental.pallas import tpu_sc as plsc

NM = 10000
E = 160000
EP = 163840
LAT = 64
NC = 2
NSUB = 16
NW = NC * NSUB
EPW = EP // NW
BLK = 256
NBLK = EPW // BLK
CHUNK = 128
NCH = BLK // CHUNK
SBLK = 128
SNB = EPW // SBLK
NPS = NM // NSUB
RN = 2000
RE = 2048
STEPS = 3
EPS = 1e-5


def _ln(y):
    m = jnp.mean(y, axis=-1, keepdims=True)
    d = y - m
    v = jnp.mean(d * d, axis=-1, keepdims=True)
    return d * lax.rsqrt(v + EPS)


def _enc_body(x_ref, w1, b1, w2, b2, w3, b3, o_ref):
    x = x_ref[0]
    h = jnp.maximum(x @ w1[0] + b1[0], 0.0)
    h = jnp.maximum(h @ w2[0] + b2[0], 0.0)
    y = h @ w3[0] + b3[0]
    o_ref[0] = _ln(y)


def _encode(x, rows, w1, b1, w2, b2, w3, b3):
    s, n, d = x.shape
    return pl.pallas_call(
        _enc_body,
        grid=(s, n // rows),
        in_specs=[
            pl.BlockSpec((1, rows, d), lambda i, b: (i, b, 0)),
            pl.BlockSpec((1, d, LAT), lambda i, b: (i, 0, 0)),
            pl.BlockSpec((1, 1, LAT), lambda i, b: (i, 0, 0)),
            pl.BlockSpec((1, LAT, LAT), lambda i, b: (i, 0, 0)),
            pl.BlockSpec((1, 1, LAT), lambda i, b: (i, 0, 0)),
            pl.BlockSpec((1, LAT, LAT), lambda i, b: (i, 0, 0)),
            pl.BlockSpec((1, 1, LAT), lambda i, b: (i, 0, 0)),
        ],
        out_specs=pl.BlockSpec((1, rows, LAT), lambda i, b: (i, b, 0)),
        out_shape=jax.ShapeDtypeStruct((s, n, LAT), jnp.float32),
    )(x, w1, b1, w2, b2, w3, b3)


def _proj_body(n_ref, w_ref, o_ref):
    x = n_ref[0]
    o_ref[0, 0] = (x @ w_ref[0, 0]).astype(jnp.bfloat16)
    o_ref[0, 1] = (x @ w_ref[0, 1]).astype(jnp.bfloat16)


def _proj(ns, wsel):
    return pl.pallas_call(
        _proj_body,
        grid=(2, NM // RN),
        in_specs=[
            pl.BlockSpec((1, RN, LAT), lambda i, b: (i, b, 0)),
            pl.BlockSpec((1, 2, LAT, LAT), lambda i, b: (i, 0, 0, 0)),
        ],
        out_specs=pl.BlockSpec((1, 2, RN, LAT), lambda i, b: (i, 0, b, 0)),
        out_shape=jax.ShapeDtypeStruct((2, 2, NM, LAT), jnp.bfloat16),
    )(ns, wsel)


def _tail_body(e_ref, gs_ref, gd_ref, w1, b1, w2, b2, w3, b3, o_ref):
    e = e_ref[0]
    x = (e @ w1[0] + gs_ref[0].astype(jnp.float32)
         + gd_ref[0].astype(jnp.float32) + b1[0])
    h = jnp.maximum(x, 0.0)
    h = jnp.maximum(h @ w2[0] + b2[0], 0.0)
    y = h @ w3[0] + b3[0]
    out = e + _ln(y)
    rows = pl.program_id(1) * RE + lax.broadcasted_iota(jnp.int32, (RE, 1), 0)
    o_ref[0] = jnp.where(rows < E, out, 0.0)


def _tail(e, gs, gd, w1, b1, w2, b2, w3, b3):
    wspec = pl.BlockSpec((1, LAT, LAT), lambda i, b: (i, 0, 0))
    bspec = pl.BlockSpec((1, 1, LAT), lambda i, b: (i, 0, 0))
    espec = pl.BlockSpec((1, RE, LAT), lambda i, b: (i, b, 0))
    return pl.pallas_call(
        _tail_body,
        grid=(2, EP // RE),
        in_specs=[espec, espec, espec, wspec, bspec, wspec, bspec, wspec, bspec],
        out_specs=espec,
        out_shape=jax.ShapeDtypeStruct((2, EP, LAT), jnp.float32),
    )(e, gs, gd, w1, b1, w2, b2, w3, b3)


def _nupd_body(n_ref, a_ref, w1a, w1b, b1, w2, b2, w3, b3, o_ref):
    x = n_ref[0]
    agg = a_ref[0, 0] + a_ref[0, 1]
    h = jnp.maximum(x @ w1a[0] + agg @ w1b[0] + b1[0], 0.0)
    h = jnp.maximum(h @ w2[0] + b2[0], 0.0)
    y = h @ w3[0] + b3[0]
    o_ref[0] = x + _ln(y)


def _nupd(ns, aggs, w1a, w1b, b1, w2, b2, w3, b3):
    wspec = pl.BlockSpec((1, LAT, LAT), lambda i, b: (i, 0, 0))
    bspec = pl.BlockSpec((1, 1, LAT), lambda i, b: (i, 0, 0))
    return pl.pallas_call(
        _nupd_body,
        grid=(2, NM // RN),
        in_specs=[
            pl.BlockSpec((1, RN, LAT), lambda i, b: (i, b, 0)),
            pl.BlockSpec((1, 2, RN, LAT), lambda i, b: (i, 0, b, 0)),
            wspec, wspec, bspec, wspec, bspec, wspec, bspec,
        ],
        out_specs=pl.BlockSpec((1, RN, LAT), lambda i, b: (i, b, 0)),
        out_shape=jax.ShapeDtypeStruct((2, NM, LAT), jnp.float32),
    )(ns, aggs, w1a, w1b, b1, w2, b2, w3, b3)


def _dec_body(n_ref, w1, b1, w2, b2, w3, b3, o_ref):
    x = n_ref[0]
    h = jnp.maximum(x @ w1[0] + b1[0], 0.0)
    h = jnp.maximum(h @ w2[0] + b2[0], 0.0)
    o_ref[0] = h @ w3[0] + b3[0]


def _dec(ns, w1, b1, w2, b2, w3, b3):
    wspec = pl.BlockSpec((1, LAT, LAT), lambda i, b: (i, 0, 0))
    bspec = pl.BlockSpec((1, 1, LAT), lambda i, b: (i, 0, 0))
    return pl.pallas_call(
        _dec_body,
        grid=(2, NM // RN),
        in_specs=[
            pl.BlockSpec((1, RN, LAT), lambda i, b: (i, b, 0)),
            wspec, bspec, wspec, bspec,
            pl.BlockSpec((1, LAT, 3), lambda i, b: (i, 0, 0)),
            pl.BlockSpec((1, 1, 3), lambda i, b: (i, 0, 0)),
        ],
        out_specs=pl.BlockSpec((1, RN, 3), lambda i, b: (i, b, 0)),
        out_shape=jax.ShapeDtypeStruct((2, NM, 3), jnp.float32),
    )(ns, w1, b1, w2, b2, w3, b3)


@functools.cache
def _sc_kernels():
    mesh = plsc.VectorSubcoreMesh(core_axis_name="c", subcore_axis_name="s",
                                  num_cores=NC, num_subcores=NSUB)

    @functools.partial(
        pl.kernel,
        out_type=(
            jax.ShapeDtypeStruct((2 * EP, LAT), jnp.bfloat16),
            jax.ShapeDtypeStruct((2 * EP, LAT), jnp.bfloat16),
        ),
        mesh=mesh,
        scratch_types=[
            pltpu.VMEM((4, NBLK * NCH, CHUNK), jnp.int32),
            pltpu.VMEM((BLK, LAT), jnp.bfloat16),
            pltpu.VMEM((BLK, LAT), jnp.bfloat16),
            pltpu.VMEM((BLK, LAT), jnp.bfloat16),
            pltpu.VMEM((BLK, LAT), jnp.bfloat16),
            pltpu.SemaphoreType.DMA,
            pltpu.SemaphoreType.DMA,
        ],
        compiler_params=pltpu.CompilerParams(use_tc_tiling_on_sc=False),
    )
    def _sc_gather(t00, t10, t11, t01, gidx, gs_out, gd_out,
                   idx_v, ba0, bb0, ba1, bb1, sem0, sem1):
        c = lax.axis_index("c")
        s = lax.axis_index("s")
        wid = s * NC + c
        pltpu.sync_copy(gidx.at[wid], idx_v)
        bufs = ((ba0, bb0, sem0), (ba1, bb1, sem1))

        for t, tsrc, tdst, rs, rd in ((0, t00, t10, 0, 1),
                                      (1, t11, t01, 2, 3)):
            def descs(b, par):
                ba, bb_, sem = bufs[par]
                out = []
                for cc in range(NCH):
                    j = b * NCH + cc
                    out.append(pltpu.make_async_copy(
                        tsrc.at[idx_v.at[rs, j]],
                        ba.at[pl.ds(cc * CHUNK, CHUNK)], sem))
                    out.append(pltpu.make_async_copy(
                        tdst.at[idx_v.at[rd, j]],
                        bb_.at[pl.ds(cc * CHUNK, CHUNK)], sem))
                return out

            def g_start(b, par):
                for d in descs(b, par):
                    d.start()

            def g_wait(b, par):
                for d in descs(b, par):
                    d.wait()

            def wr(b, par):
                ba, bb_, _ = bufs[par]
                base = wid * EPW + b * BLK
                pltpu.sync_copy(ba, gs_out.at[pl.ds(t * EP + base, BLK)])
                pltpu.sync_copy(bb_, gd_out.at[pl.ds(t * EP + base, BLK)])

            g_start(0, 0)

            def body(bb, carry):
                b0 = 2 * bb
                g_start(b0 + 1, 1)
                g_wait(b0, 0)
                wr(b0, 0)
                pl.when(bb < NBLK // 2 - 1)(lambda: g_start(b0 + 2, 0))
                g_wait(b0 + 1, 1)
                wr(b0 + 1, 1)
                return carry

            lax.fori_loop(0, NBLK // 2, body, 0)

    @functools.partial(
        pl.kernel,
        out_type=jax.ShapeDtypeStruct((4 * NM, LAT), jnp.float32),
        mesh=mesh,
        scratch_types=[
            pltpu.VMEM((2, SNB, CHUNK), jnp.int32),
            pltpu.VMEM((SBLK, LAT), jnp.float32),
            pltpu.VMEM((SBLK, LAT), jnp.float32),
            pltpu.VMEM((SBLK, LAT), jnp.float32),
            pltpu.VMEM((SBLK, LAT), jnp.float32),
            pltpu.VMEM_SHARED((NM, LAT), jnp.float32),
            pltpu.VMEM_SHARED((NM, LAT), jnp.float32),
            pltpu.SemaphoreType.DMA,
            pltpu.SemaphoreType.DMA,
            pltpu.SemaphoreType.DMA,
            pltpu.SemaphoreType.DMA,
            pltpu.SemaphoreType.DMA,
            pltpu.SemaphoreType.DMA,
            pltpu.SemaphoreType.DMA,
            pltpu.SemaphoreType.DMA,
        ],
        compiler_params=pltpu.CompilerParams(use_tc_tiling_on_sc=False),
    )
    def _sc_scatter(enew, sidx, zeros, agg_out, idx_v, r0, r1, r2, r3,
                    spm0, spm1, l0, l1, l2, l3, a0, a1, a2, a3):
        c = lax.axis_index("c")
        s = lax.axis_index("s")
        wid = s * NC + c
        rows = (r0, r1, r2, r3)
        lsems = (l0, l1, l2, l3)
        asems = (a0, a1, a2, a3)
        pltpu.sync_copy(zeros.at[pl.ds(s * NPS, NPS)],
                        spm0.at[pl.ds(s * NPS, NPS)])
        pltpu.sync_copy(zeros.at[pl.ds(s * NPS, NPS)],
                        spm1.at[pl.ds(s * NPS, NPS)])
        pltpu.sync_copy(sidx.at[wid], idx_v)
        plsc.subcore_barrier()

        for t, spm in ((0, spm0), (1, spm1)):
            def load_desc(k, b):
                base = wid * EPW + b * SBLK
                return pltpu.make_async_copy(
                    enew.at[pl.ds(t * EP + base, SBLK)], rows[k], lsems[k])

            def add_desc(k, b):
                return pltpu.make_async_copy(
                    rows[k], spm.at[idx_v.at[t, b]], asems[k])

            for k in range(4):
                load_desc(k, k).start()

            def body(bb, carry):
                for k in range(4):
                    b = 4 * bb + k
                    load_desc(k, b).wait()
                    add_desc(k, b).start(add=True)
                for k in range(4):
                    b = 4 * bb + k

                    def refill(k=k, b=b):
                        add_desc(k, b).wait()
                        load_desc(k, b + 4).start()

                    pl.when(bb < SNB // 4 - 1)(refill)
                return carry

            lax.fori_loop(0, SNB // 4, body, 0)
            for k in range(4):
                add_desc(k, SNB - 4 + k).wait()
        plsc.subcore_barrier()
        off = s * NPS
        pltpu.sync_copy(spm0.at[pl.ds(off, NPS)],
                        agg_out.at[pl.ds((2 + c) * NM + off, NPS)])
        pltpu.sync_copy(spm1.at[pl.ds(off, NPS)],
                        agg_out.at[pl.ds(c * NM + off, NPS)])

    return _sc_gather, _sc_scatter


def _fold_first_layer(ps, mean, std):
    (w1, b1), (w2, b2), (w3, b3) = ps
    w1f = w1 / std[:, None]
    b1f = b1 - (mean / std) @ w1
    return w1f, b1f, w2, b2, w3, b3


def _stack_mlp(a, b):
    out = []
    for x, y in zip(a, b):
        st = jnp.stack([x, y])
        if st.ndim == 2:
            st = st[:, None, :]
        out.append(st)
    return out


def kernel(mesh_position, mesh_properties, mesh_kinematic, obj_position,
           obj_properties, obj_kinematic, om_index, om_attr, mo_index,
           mo_attr, params):
    p = params
    m_x = jnp.concatenate([
        mesh_position[1] - mesh_position[0],
        mesh_position[2] - mesh_position[1],
        mesh_properties, mesh_kinematic], axis=-1)
    o_x = jnp.concatenate([
        obj_position[1] - obj_position[0],
        obj_position[2] - obj_position[1],
        obj_properties, obj_kinematic], axis=-1)
    xn = jnp.stack([o_x, m_x])

    ns = _encode(xn, RN, *_stack_mlp(
        _fold_first_layer(p['obj_enc'], p['node_mean'], p['node_std']),
        _fold_first_layer(p['mesh_enc'], p['node_mean'], p['node_std'])))

    pad2 = ((0, EP - E), (0, 0))
    ea = jnp.stack([jnp.pad(om_attr, pad2), jnp.pad(mo_attr, pad2)])
    e = _encode(ea, RE, *_stack_mlp(
        _fold_first_layer(p['om_edge_enc'], p['om_mean'], p['om_std']),
        _fold_first_layer(p['mo_edge_enc'], p['mo_mean'], p['mo_std'])))

    pad1 = lambda v: jnp.pad(v.astype(jnp.int32), (0, EP - E))
    gidx = jnp.stack([pad1(om_index[0]), pad1(om_index[1]),
                      pad1(mo_index[0]), pad1(mo_index[1])])
    gidx = gidx.reshape(4, NW, NBLK * NCH, CHUNK).transpose(1, 0, 2, 3)
    sidx = gidx[:, (1, 3)]
    zeros = jnp.zeros((NM, LAT), jnp.float32)
    sc_gather, sc_scatter = _sc_kernels()

    for st in range(STEPS):
        sp = p['steps'][st]
        om_w1, om_b1 = sp['om_e'][0]
        mo_w1, mo_b1 = sp['mo_e'][0]
        wsel = jnp.stack([
            jnp.stack([om_w1[LAT:2 * LAT], mo_w1[2 * LAT:]]),
            jnp.stack([om_w1[2 * LAT:], mo_w1[LAT:2 * LAT]]),
        ])
        tbl = _proj(ns, wsel)
        gs_f, gd_f = sc_gather(tbl[0, 0], tbl[1, 0], tbl[1, 1], tbl[0, 1],
                               gidx)

        ew = _stack_mlp(
            (om_w1[:LAT], om_b1, sp['om_e'][1][0], sp['om_e'][1][1],
             sp['om_e'][2][0], sp['om_e'][2][1]),
            (mo_w1[:LAT], mo_b1, sp['mo_e'][1][0], sp['mo_e'][1][1],
             sp['mo_e'][2][0], sp['mo_e'][2][1]))
        e = _tail(e, gs_f.reshape(2, EP, LAT), gd_f.reshape(2, EP, LAT), *ew)

        aggs = sc_scatter(e.reshape(2 * EP, LAT), sidx, zeros)
        aggs = aggs.reshape(2, 2, NM, LAT)

        ob_w1, ob_b1 = sp['obj_n'][0]
        me_w1, me_b1 = sp['mesh_n'][0]
        nw = _stack_mlp(
            (ob_w1[:LAT], ob_b1, sp['obj_n'][1][0], sp['obj_n'][1][1],
             sp['obj_n'][2][0], sp['obj_n'][2][1]),
            (me_w1[:LAT], me_b1, sp['mesh_n'][1][0], sp['mesh_n'][1][1],
             sp['mesh_n'][2][0], sp['mesh_n'][2][1]))
        w1b = jnp.stack([ob_w1[LAT:], me_w1[LAT:]])
        ns = _nupd(ns, aggs, nw[0], w1b, *nw[1:])

    acc = _dec(ns, *_stack_mlp(
        (p['obj_dec'][0][0], p['obj_dec'][0][1], p['obj_dec'][1][0],
         p['obj_dec'][1][1], p['obj_dec'][2][0], p['obj_dec'][2][1]),
        (p['mesh_dec'][0][0], p['mesh_dec'][0][1], p['mesh_dec'][1][0],
         p['mesh_dec'][1][1], p['mesh_dec'][2][0], p['mesh_dec'][2][1])))
    return acc[1], acc[0]


# speedup vs baseline: 1.0662x; 1.0662x over previous
import functools

import jax
import jax.numpy as jnp
from jax import lax
from jax.experimental import pallas as pl
from jax.experimental.pallas import tpu as pltpu
from jax.experimental.pallas import tpu_sc as plsc

NM = 10000
E = 160000
EP = 163840
LAT = 64
NC = 2
NSUB = 16
NW = NC * NSUB
EPW = EP // NW
CHUNK = 128
NCHTOT = EPW // CHUNK
GBLK = 256
GNB = EPW // GBLK
GNCH = GBLK // CHUNK
SBLK = 256
SNB = EPW // SBLK
SNCH = SBLK // CHUNK
NPS = NM // NSUB
RN = 2000
RE = 2048
STEPS = 3
EPS = 1e-5


def _ln(y):
    m = jnp.mean(y, axis=-1, keepdims=True)
    d = y - m
    v = jnp.mean(d * d, axis=-1, keepdims=True)
    return d * lax.rsqrt(v + EPS)


def _enc_body(x_ref, w1, b1, w2, b2, w3, b3, o_ref):
    x = x_ref[...]
    h = jnp.maximum(x @ w1[...] + b1[...], 0.0)
    h = jnp.maximum(h @ w2[...] + b2[...], 0.0)
    y = h @ w3[...] + b3[...]
    o_ref[...] = _ln(y)


def _full(shape):
    return pl.BlockSpec(shape, lambda b: (0,) * len(shape))


def _encode(x, rows, w1, b1, w2, b2, w3, b3):
    n, d = x.shape
    return pl.pallas_call(
        _enc_body,
        grid=(n // rows,),
        in_specs=[
            pl.BlockSpec((rows, d), lambda b: (b, 0)),
            _full((d, LAT)), _full((1, LAT)),
            _full((LAT, LAT)), _full((1, LAT)),
            _full((LAT, LAT)), _full((1, LAT)),
        ],
        out_specs=pl.BlockSpec((rows, LAT), lambda b: (b, 0)),
        out_shape=jax.ShapeDtypeStruct((n, LAT), jnp.float32),
    )(x, w1, b1, w2, b2, w3, b3)


def _proj_body(n_ref, wa, wb, oa_ref, ob_ref):
    x = n_ref[...]
    oa_ref[...] = x @ wa[...]
    ob_ref[...] = x @ wb[...]


def _proj(ns, wa, wb):
    spec = pl.BlockSpec((RN, LAT), lambda b: (b, 0))
    return pl.pallas_call(
        _proj_body,
        grid=(NM // RN,),
        in_specs=[spec, _full((LAT, LAT)), _full((LAT, LAT))],
        out_specs=[spec, spec],
        out_shape=[jax.ShapeDtypeStruct((NM, LAT), jnp.float32)] * 2,
    )(ns, wa, wb)


def _tail_body(e_ref, gs_ref, gd_ref, w1, b1, w2, b2, w3, b3, o_ref):
    e = e_ref[...]
    x = e @ w1[...] + gs_ref[...] + gd_ref[...] + b1[...]
    h = jnp.maximum(x, 0.0)
    h = jnp.maximum(h @ w2[...] + b2[...], 0.0)
    y = h @ w3[...] + b3[...]
    out = e + _ln(y)
    rows = pl.program_id(0) * RE + lax.broadcasted_iota(jnp.int32, (RE, 1), 0)
    o_ref[...] = jnp.where(rows < E, out, 0.0)


def _tail(e, gs, gd, w1, b1, w2, b2, w3, b3):
    espec = pl.BlockSpec((RE, LAT), lambda b: (b, 0))
    return pl.pallas_call(
        _tail_body,
        grid=(EP // RE,),
        in_specs=[espec, espec, espec,
                  _full((LAT, LAT)), _full((1, LAT)),
                  _full((LAT, LAT)), _full((1, LAT)),
                  _full((LAT, LAT)), _full((1, LAT))],
        out_specs=espec,
        out_shape=jax.ShapeDtypeStruct((EP, LAT), jnp.float32),
    )(e, gs, gd, w1, b1, w2, b2, w3, b3)


def _nupd_body(n_ref, a_ref, w1a, w1b, b1, w2, b2, w3, b3, o_ref):
    x = n_ref[...]
    agg = a_ref[0] + a_ref[1]
    h = jnp.maximum(x @ w1a[...] + agg @ w1b[...] + b1[...], 0.0)
    h = jnp.maximum(h @ w2[...] + b2[...], 0.0)
    y = h @ w3[...] + b3[...]
    o_ref[...] = x + _ln(y)


def _nupd(ns, agg2, w1a, w1b, b1, w2, b2, w3, b3):
    spec = pl.BlockSpec((RN, LAT), lambda b: (b, 0))
    return pl.pallas_call(
        _nupd_body,
        grid=(NM // RN,),
        in_specs=[spec,
                  pl.BlockSpec((2, RN, LAT), lambda b: (0, b, 0)),
                  _full((LAT, LAT)), _full((LAT, LAT)), _full((1, LAT)),
                  _full((LAT, LAT)), _full((1, LAT)),
                  _full((LAT, LAT)), _full((1, LAT))],
        out_specs=spec,
        out_shape=jax.ShapeDtypeStruct((NM, LAT), jnp.float32),
    )(ns, agg2, w1a, w1b, b1, w2, b2, w3, b3)


def _dec_body(n_ref, w1, b1, w2, b2, w3, b3, o_ref):
    x = n_ref[...]
    h = jnp.maximum(x @ w1[...] + b1[...], 0.0)
    h = jnp.maximum(h @ w2[...] + b2[...], 0.0)
    o_ref[...] = h @ w3[...] + b3[...]


def _dec(ns, w1, b1, w2, b2, w3, b3):
    return pl.pallas_call(
        _dec_body,
        grid=(NM // RN,),
        in_specs=[pl.BlockSpec((RN, LAT), lambda b: (b, 0)),
                  _full((LAT, LAT)), _full((1, LAT)),
                  _full((LAT, LAT)), _full((1, LAT)),
                  _full((LAT, 3)), _full((1, 3))],
        out_specs=pl.BlockSpec((RN, 3), lambda b: (b, 0)),
        out_shape=jax.ShapeDtypeStruct((NM, 3), jnp.float32),
    )(ns, w1, b1, w2, b2, w3, b3)


@functools.cache
def _sc_kernels():
    mesh = plsc.VectorSubcoreMesh(core_axis_name="c", subcore_axis_name="s",
                                  num_cores=NC, num_subcores=NSUB)

    @functools.partial(
        pl.kernel,
        out_type=(
            jax.ShapeDtypeStruct((EP, LAT), jnp.float32),
            jax.ShapeDtypeStruct((EP, LAT), jnp.float32),
        ),
        mesh=mesh,
        scratch_types=[
            pltpu.VMEM((2, NCHTOT, CHUNK), jnp.int32),
            pltpu.VMEM((GBLK, LAT), jnp.float32),
            pltpu.VMEM((GBLK, LAT), jnp.float32),
            pltpu.VMEM((GBLK, LAT), jnp.float32),
            pltpu.VMEM((GBLK, LAT), jnp.float32),
            pltpu.SemaphoreType.DMA,
            pltpu.SemaphoreType.DMA,
        ],
        compiler_params=pltpu.CompilerParams(use_tc_tiling_on_sc=False),
    )
    def _sc_gather(tsrc, tdst, gidx, gs_out, gd_out,
                   idx_v, ba0, bb0, ba1, bb1, sem0, sem1):
        c = lax.axis_index("c")
        s = lax.axis_index("s")
        wid = s * NC + c
        pltpu.sync_copy(gidx.at[wid], idx_v)
        bufs = ((ba0, bb0, sem0), (ba1, bb1, sem1))

        def descs(b, par):
            ba, bb_, sem = bufs[par]
            out = []
            for cc in range(GNCH):
                j = b * GNCH + cc
                out.append(pltpu.make_async_copy(
                    tsrc.at[idx_v.at[0, j]],
                    ba.at[pl.ds(cc * CHUNK, CHUNK)], sem))
                out.append(pltpu.make_async_copy(
                    tdst.at[idx_v.at[1, j]],
                    bb_.at[pl.ds(cc * CHUNK, CHUNK)], sem))
            return out

        def g_start(b, par):
            for d in descs(b, par):
                d.start()

        def g_wait(b, par):
            for d in descs(b, par):
                d.wait()

        def wr(b, par):
            ba, bb_, _ = bufs[par]
            base = wid * EPW + b * GBLK
            pltpu.sync_copy(ba, gs_out.at[pl.ds(base, GBLK)])
            pltpu.sync_copy(bb_, gd_out.at[pl.ds(base, GBLK)])

        g_start(0, 0)

        def body(bb, carry):
            b0 = 2 * bb
            g_start(b0 + 1, 1)
            g_wait(b0, 0)
            wr(b0, 0)
            pl.when(bb < GNB // 2 - 1)(lambda: g_start(b0 + 2, 0))
            g_wait(b0 + 1, 1)
            wr(b0 + 1, 1)
            return carry

        lax.fori_loop(0, GNB // 2, body, 0)

    @functools.partial(
        pl.kernel,
        out_type=jax.ShapeDtypeStruct((2 * NM, LAT), jnp.float32),
        mesh=mesh,
        scratch_types=[
            pltpu.VMEM((NCHTOT, CHUNK), jnp.int32),
            pltpu.VMEM((SBLK, LAT), jnp.float32),
            pltpu.VMEM((SBLK, LAT), jnp.float32),
            pltpu.VMEM((SBLK, LAT), jnp.float32),
            pltpu.VMEM((SBLK, LAT), jnp.float32),
            pltpu.VMEM_SHARED((NM, LAT), jnp.float32),
            pltpu.SemaphoreType.DMA,
            pltpu.SemaphoreType.DMA,
            pltpu.SemaphoreType.DMA,
            pltpu.SemaphoreType.DMA,
            pltpu.SemaphoreType.DMA,
            pltpu.SemaphoreType.DMA,
            pltpu.SemaphoreType.DMA,
            pltpu.SemaphoreType.DMA,
        ],
        compiler_params=pltpu.CompilerParams(use_tc_tiling_on_sc=False),
    )
    def _sc_scatter(enew, sidx, zeros, agg_out, idx_v, r0, r1, r2, r3,
                    spm, l0, l1, l2, l3, a0, a1, a2, a3):
        c = lax.axis_index("c")
        s = lax.axis_index("s")
        wid = s * NC + c
        rows = (r0, r1, r2, r3)
        lsems = (l0, l1, l2, l3)
        asems = (a0, a1, a2, a3)
        pltpu.sync_copy(zeros.at[pl.ds(s * NPS, NPS)],
                        spm.at[pl.ds(s * NPS, NPS)])
        pltpu.sync_copy(sidx.at[wid], idx_v)
        plsc.subcore_barrier()

        def load_desc(k, b):
            base = wid * EPW + b * SBLK
            return pltpu.make_async_copy(
                enew.at[pl.ds(base, SBLK)], rows[k], lsems[k])

        def add_descs(k, b):
            out = []
            for cc in range(SNCH):
                j = b * SNCH + cc
                out.append(pltpu.make_async_copy(
                    rows[k].at[pl.ds(cc * CHUNK, CHUNK)],
                    spm.at[idx_v.at[j]], asems[k]))
            return out

        for k in range(4):
            load_desc(k, k).start()

        def body(bb, carry):
            for k in range(4):
                b = 4 * bb + k
                load_desc(k, b).wait()
                for d in add_descs(k, b):
                    d.start(add=True)
            for k in range(4):
                b = 4 * bb + k

                def refill(k=k, b=b):
                    for d in add_descs(k, b):
                        d.wait()
                    load_desc(k, b + 4).start()

                pl.when(bb < SNB // 4 - 1)(refill)
            return carry

        lax.fori_loop(0, SNB // 4, body, 0)
        for k in range(4):
            for d in add_descs(k, SNB - 4 + k):
                d.wait()
        plsc.subcore_barrier()
        off = s * NPS
        pltpu.sync_copy(spm.at[pl.ds(off, NPS)],
                        agg_out.at[pl.ds(c * NM + off, NPS)])

    return _sc_gather, _sc_scatter


def _fold_first_layer(ps, mean, std):
    (w1, b1), (w2, b2), (w3, b3) = ps
    w1f = w1 / std[:, None]
    b1f = b1 - (mean / std) @ w1
    return w1f, b1f[None], w2, b2[None], w3, b3[None]


def _row_biases(ps):
    out = []
    for w, b in ps:
        out.append(w)
        out.append(b[None])
    return out


def _widx(v):
    return jnp.pad(v.astype(jnp.int32), (0, EP - E)).reshape(
        NW, NCHTOT, CHUNK)


def kernel(mesh_position, mesh_properties, mesh_kinematic, obj_position,
           obj_properties, obj_kinematic, om_index, om_attr, mo_index,
           mo_attr, params):
    p = params
    m_x = jnp.concatenate([
        mesh_position[1] - mesh_position[0],
        mesh_position[2] - mesh_position[1],
        mesh_properties, mesh_kinematic], axis=-1)
    o_x = jnp.concatenate([
        obj_position[1] - obj_position[0],
        obj_position[2] - obj_position[1],
        obj_properties, obj_kinematic], axis=-1)

    ns_o = _encode(o_x, RN, *_fold_first_layer(
        p['obj_enc'], p['node_mean'], p['node_std']))
    ns_m = _encode(m_x, RN, *_fold_first_layer(
        p['mesh_enc'], p['node_mean'], p['node_std']))

    pad2 = ((0, EP - E), (0, 0))
    e_om = _encode(jnp.pad(om_attr, pad2), RE, *_fold_first_layer(
        p['om_edge_enc'], p['om_mean'], p['om_std']))
    e_mo = _encode(jnp.pad(mo_attr, pad2), RE, *_fold_first_layer(
        p['mo_edge_enc'], p['mo_mean'], p['mo_std']))

    gidx_om = jnp.stack([_widx(om_index[0]), _widx(om_index[1])], axis=1)
    gidx_mo = jnp.stack([_widx(mo_index[0]), _widx(mo_index[1])], axis=1)
    sidx_om = _widx(om_index[1])
    sidx_mo = _widx(mo_index[1])
    zeros = jnp.zeros((NM, LAT), jnp.float32)
    sc_gather, sc_scatter = _sc_kernels()

    for st in range(STEPS):
        sp = p['steps'][st]
        om_w1, om_b1 = sp['om_e'][0]
        mo_w1, mo_b1 = sp['mo_e'][0]
        t_om_src, t_mo_dst = _proj(ns_o, om_w1[LAT:2 * LAT], mo_w1[2 * LAT:])
        t_om_dst, t_mo_src = _proj(ns_m, om_w1[2 * LAT:], mo_w1[LAT:2 * LAT])

        g_om_s, g_om_d = sc_gather(t_om_src, t_om_dst, gidx_om)
        g_mo_s, g_mo_d = sc_gather(t_mo_src, t_mo_dst, gidx_mo)

        e_om = _tail(e_om, g_om_s, g_om_d, om_w1[:LAT], om_b1[None],
                     *_row_biases(sp['om_e'][1:]))
        e_mo = _tail(e_mo, g_mo_s, g_mo_d, mo_w1[:LAT], mo_b1[None],
                     *_row_biases(sp['mo_e'][1:]))

        agg_m2 = sc_scatter(e_om, sidx_om, zeros).reshape(2, NM, LAT)
        agg_o2 = sc_scatter(e_mo, sidx_mo, zeros).reshape(2, NM, LAT)

        me_w1, me_b1 = sp['mesh_n'][0]
        ob_w1, ob_b1 = sp['obj_n'][0]
        ns_m = _nupd(ns_m, agg_m2, me_w1[:LAT], me_w1[LAT:], me_b1[None],
                     *_row_biases(sp['mesh_n'][1:]))
        ns_o = _nupd(ns_o, agg_o2, ob_w1[:LAT], ob_w1[LAT:], ob_b1[None],
                     *_row_biases(sp['obj_n'][1:]))

    m_acc = _dec(ns_m, *_row_biases(p['mesh_dec']))
    o_acc = _dec(ns_o, *_row_biases(p['obj_dec']))
    return m_acc, o_acc


# speedup vs baseline: 1.2764x; 1.1972x over previous
import functools

import jax
import jax.numpy as jnp
from jax import lax
from jax.experimental import pallas as pl
from jax.experimental.pallas import tpu as pltpu
from jax.experimental.pallas import tpu_sc as plsc

NM = 10000
E = 160000
EP = 163840
LAT = 64
NC = 2
NSUB = 16
NW = NC * NSUB
EPW = EP // NW
CHUNK = 128
NCHTOT = EPW // CHUNK
GBLK = 128
GNB = EPW // GBLK
SBLK = 256
SNB = EPW // SBLK
SNCH = SBLK // CHUNK
NPS = NM // NSUB
RN = 2000
RE = 2048
STEPS = 3
EPS = 1e-5


def _ln(y):
    m = jnp.mean(y, axis=-1, keepdims=True)
    d = y - m
    v = jnp.mean(d * d, axis=-1, keepdims=True)
    return d * lax.rsqrt(v + EPS)


def _enc_body(x_ref, w1, b1, w2, b2, w3, b3, o_ref):
    x = x_ref[...]
    h = jnp.maximum(x @ w1[...] + b1[...], 0.0)
    h = jnp.maximum(h @ w2[...] + b2[...], 0.0)
    y = h @ w3[...] + b3[...]
    o_ref[...] = _ln(y)


def _full(shape):
    return pl.BlockSpec(shape, lambda b: (0,) * len(shape))


def _encode(x, rows, w1, b1, w2, b2, w3, b3):
    n, d = x.shape
    return pl.pallas_call(
        _enc_body,
        grid=(n // rows,),
        in_specs=[
            pl.BlockSpec((rows, d), lambda b: (b, 0)),
            _full((d, LAT)), _full((1, LAT)),
            _full((LAT, LAT)), _full((1, LAT)),
            _full((LAT, LAT)), _full((1, LAT)),
        ],
        out_specs=pl.BlockSpec((rows, LAT), lambda b: (b, 0)),
        out_shape=jax.ShapeDtypeStruct((n, LAT), jnp.float32),
    )(x, w1, b1, w2, b2, w3, b3)


def _proj_body(n_ref, wa, wb, oa_ref, ob_ref):
    x = n_ref[...]
    oa_ref[...] = x @ wa[...]
    ob_ref[...] = x @ wb[...]


def _proj(ns, wa, wb):
    spec = pl.BlockSpec((RN, LAT), lambda b: (b, 0))
    return pl.pallas_call(
        _proj_body,
        grid=(NM // RN,),
        in_specs=[spec, _full((LAT, LAT)), _full((LAT, LAT))],
        out_specs=[spec, spec],
        out_shape=[jax.ShapeDtypeStruct((NM, LAT), jnp.float32)] * 2,
    )(ns, wa, wb)


def _tail_body(e_ref, gs_ref, gd_ref, w1, b1, w2, b2, w3, b3, o_ref):
    e = e_ref[...]
    x = e @ w1[...] + gs_ref[...] + gd_ref[...] + b1[...]
    h = jnp.maximum(x, 0.0)
    h = jnp.maximum(h @ w2[...] + b2[...], 0.0)
    y = h @ w3[...] + b3[...]
    out = e + _ln(y)
    rows = pl.program_id(0) * RE + lax.broadcasted_iota(jnp.int32, (RE, 1), 0)
    o_ref[...] = jnp.where(rows < E, out, 0.0)


def _tail(e, gs, gd, w1, b1, w2, b2, w3, b3):
    espec = pl.BlockSpec((RE, LAT), lambda b: (b, 0))
    return pl.pallas_call(
        _tail_body,
        grid=(EP // RE,),
        in_specs=[espec, espec, espec,
                  _full((LAT, LAT)), _full((1, LAT)),
                  _full((LAT, LAT)), _full((1, LAT)),
                  _full((LAT, LAT)), _full((1, LAT))],
        out_specs=espec,
        out_shape=jax.ShapeDtypeStruct((EP, LAT), jnp.float32),
    )(e, gs, gd, w1, b1, w2, b2, w3, b3)


def _nupd_body(n_ref, a_ref, w1a, w1b, b1, w2, b2, w3, b3, o_ref):
    x = n_ref[...]
    agg = a_ref[0] + a_ref[1]
    h = jnp.maximum(x @ w1a[...] + agg @ w1b[...] + b1[...], 0.0)
    h = jnp.maximum(h @ w2[...] + b2[...], 0.0)
    y = h @ w3[...] + b3[...]
    o_ref[...] = x + _ln(y)


def _nupd(ns, agg2, w1a, w1b, b1, w2, b2, w3, b3):
    spec = pl.BlockSpec((RN, LAT), lambda b: (b, 0))
    return pl.pallas_call(
        _nupd_body,
        grid=(NM // RN,),
        in_specs=[spec,
                  pl.BlockSpec((2, RN, LAT), lambda b: (0, b, 0)),
                  _full((LAT, LAT)), _full((LAT, LAT)), _full((1, LAT)),
                  _full((LAT, LAT)), _full((1, LAT)),
                  _full((LAT, LAT)), _full((1, LAT))],
        out_specs=spec,
        out_shape=jax.ShapeDtypeStruct((NM, LAT), jnp.float32),
    )(ns, agg2, w1a, w1b, b1, w2, b2, w3, b3)


def _dec_body(n_ref, w1, b1, w2, b2, w3, b3, o_ref):
    x = n_ref[...]
    h = jnp.maximum(x @ w1[...] + b1[...], 0.0)
    h = jnp.maximum(h @ w2[...] + b2[...], 0.0)
    o_ref[...] = h @ w3[...] + b3[...]


def _dec(ns, w1, b1, w2, b2, w3, b3):
    return pl.pallas_call(
        _dec_body,
        grid=(NM // RN,),
        in_specs=[pl.BlockSpec((RN, LAT), lambda b: (b, 0)),
                  _full((LAT, LAT)), _full((1, LAT)),
                  _full((LAT, LAT)), _full((1, LAT)),
                  _full((LAT, 3)), _full((1, 3))],
        out_specs=pl.BlockSpec((RN, 3), lambda b: (b, 0)),
        out_shape=jax.ShapeDtypeStruct((NM, 3), jnp.float32),
    )(ns, w1, b1, w2, b2, w3, b3)


@functools.cache
def _sc_kernels():
    mesh = plsc.VectorSubcoreMesh(core_axis_name="c", subcore_axis_name="s",
                                  num_cores=NC, num_subcores=NSUB)

    @functools.partial(
        pl.kernel,
        out_type=(
            jax.ShapeDtypeStruct((EP, LAT), jnp.float32),
            jax.ShapeDtypeStruct((EP, LAT), jnp.float32),
        ),
        mesh=mesh,
        scratch_types=[
            pltpu.VMEM((2, NCHTOT, CHUNK), jnp.int32),
            pltpu.VMEM((GBLK, LAT), jnp.float32),
            pltpu.VMEM((GBLK, LAT), jnp.float32),
            pltpu.VMEM((GBLK, LAT), jnp.float32),
            pltpu.VMEM((GBLK, LAT), jnp.float32),
            pltpu.VMEM_SHARED((NM, LAT), jnp.float32),
            pltpu.VMEM_SHARED((NM, LAT), jnp.float32),
            pltpu.SemaphoreType.DMA,
            pltpu.SemaphoreType.DMA,
        ],
        compiler_params=pltpu.CompilerParams(use_tc_tiling_on_sc=False),
    )
    def _sc_gather(tsrc, tdst, gidx, gs_out, gd_out,
                   idx_v, ba0, bb0, ba1, bb1, spms, spmd, sem0, sem1):
        c = lax.axis_index("c")
        s = lax.axis_index("s")
        wid = s * NC + c
        pltpu.sync_copy(gidx.at[wid], idx_v)
        off = s * NPS
        pltpu.sync_copy(tsrc.at[pl.ds(off, NPS)], spms.at[pl.ds(off, NPS)])
        pltpu.sync_copy(tdst.at[pl.ds(off, NPS)], spmd.at[pl.ds(off, NPS)])
        plsc.subcore_barrier()
        bufs = ((ba0, bb0, sem0), (ba1, bb1, sem1))

        def descs(b, par):
            ba, bb_, sem = bufs[par]
            return [
                pltpu.make_async_copy(spms.at[idx_v.at[0, b]], ba, sem),
                pltpu.make_async_copy(spmd.at[idx_v.at[1, b]], bb_, sem),
            ]

        def g_start(b, par):
            for d in descs(b, par):
                d.start()

        def g_wait(b, par):
            for d in descs(b, par):
                d.wait()

        def wr(b, par):
            ba, bb_, _ = bufs[par]
            base = wid * EPW + b * GBLK
            pltpu.sync_copy(ba, gs_out.at[pl.ds(base, GBLK)])
            pltpu.sync_copy(bb_, gd_out.at[pl.ds(base, GBLK)])

        g_start(0, 0)

        def body(bb, carry):
            b0 = 2 * bb
            g_start(b0 + 1, 1)
            g_wait(b0, 0)
            wr(b0, 0)
            pl.when(bb < GNB // 2 - 1)(lambda: g_start(b0 + 2, 0))
            g_wait(b0 + 1, 1)
            wr(b0 + 1, 1)
            return carry

        lax.fori_loop(0, GNB // 2, body, 0)

    @functools.partial(
        pl.kernel,
        out_type=jax.ShapeDtypeStruct((2 * NM, LAT), jnp.float32),
        mesh=mesh,
        scratch_types=[
            pltpu.VMEM((NCHTOT, CHUNK), jnp.int32),
            pltpu.VMEM((SBLK, LAT), jnp.float32),
            pltpu.VMEM((SBLK, LAT), jnp.float32),
            pltpu.VMEM((SBLK, LAT), jnp.float32),
            pltpu.VMEM((SBLK, LAT), jnp.float32),
            pltpu.VMEM_SHARED((NM, LAT), jnp.float32),
            pltpu.SemaphoreType.DMA,
            pltpu.SemaphoreType.DMA,
            pltpu.SemaphoreType.DMA,
            pltpu.SemaphoreType.DMA,
            pltpu.SemaphoreType.DMA,
            pltpu.SemaphoreType.DMA,
            pltpu.SemaphoreType.DMA,
            pltpu.SemaphoreType.DMA,
        ],
        compiler_params=pltpu.CompilerParams(use_tc_tiling_on_sc=False),
    )
    def _sc_scatter(enew, sidx, zeros, agg_out, idx_v, r0, r1, r2, r3,
                    spm, l0, l1, l2, l3, a0, a1, a2, a3):
        c = lax.axis_index("c")
        s = lax.axis_index("s")
        wid = s * NC + c
        rows = (r0, r1, r2, r3)
        lsems = (l0, l1, l2, l3)
        asems = (a0, a1, a2, a3)
        pltpu.sync_copy(zeros.at[pl.ds(s * NPS, NPS)],
                        spm.at[pl.ds(s * NPS, NPS)])
        pltpu.sync_copy(sidx.at[wid], idx_v)
        plsc.subcore_barrier()

        def load_desc(k, b):
            base = wid * EPW + b * SBLK
            return pltpu.make_async_copy(
                enew.at[pl.ds(base, SBLK)], rows[k], lsems[k])

        def add_descs(k, b):
            out = []
            for cc in range(SNCH):
                j = b * SNCH + cc
                out.append(pltpu.make_async_copy(
                    rows[k].at[pl.ds(cc * CHUNK, CHUNK)],
                    spm.at[idx_v.at[j]], asems[k]))
            return out

        for k in range(4):
            load_desc(k, k).start()

        def body(bb, carry):
            for k in range(4):
                b = 4 * bb + k
                load_desc(k, b).wait()
                for d in add_descs(k, b):
                    d.start(add=True)
            for k in range(4):
                b = 4 * bb + k

                def refill(k=k, b=b):
                    for d in add_descs(k, b):
                        d.wait()
                    load_desc(k, b + 4).start()

                pl.when(bb < SNB // 4 - 1)(refill)
            return carry

        lax.fori_loop(0, SNB // 4, body, 0)
        for k in range(4):
            for d in add_descs(k, SNB - 4 + k):
                d.wait()
        plsc.subcore_barrier()
        off = s * NPS
        pltpu.sync_copy(spm.at[pl.ds(off, NPS)],
                        agg_out.at[pl.ds(c * NM + off, NPS)])

    return _sc_gather, _sc_scatter


def _fold_first_layer(ps, mean, std):
    (w1, b1), (w2, b2), (w3, b3) = ps
    w1f = w1 / std[:, None]
    b1f = b1 - (mean / std) @ w1
    return w1f, b1f[None], w2, b2[None], w3, b3[None]


def _row_biases(ps):
    out = []
    for w, b in ps:
        out.append(w)
        out.append(b[None])
    return out


def _widx(v):
    return jnp.pad(v.astype(jnp.int32), (0, EP - E)).reshape(
        NW, NCHTOT, CHUNK)


def kernel(mesh_position, mesh_properties, mesh_kinematic, obj_position,
           obj_properties, obj_kinematic, om_index, om_attr, mo_index,
           mo_attr, params):
    p = params
    m_x = jnp.concatenate([
        mesh_position[1] - mesh_position[0],
        mesh_position[2] - mesh_position[1],
        mesh_properties, mesh_kinematic], axis=-1)
    o_x = jnp.concatenate([
        obj_position[1] - obj_position[0],
        obj_position[2] - obj_position[1],
        obj_properties, obj_kinematic], axis=-1)

    ns_o = _encode(o_x, RN, *_fold_first_layer(
        p['obj_enc'], p['node_mean'], p['node_std']))
    ns_m = _encode(m_x, RN, *_fold_first_layer(
        p['mesh_enc'], p['node_mean'], p['node_std']))

    pad2 = ((0, EP - E), (0, 0))
    e_om = _encode(jnp.pad(om_attr, pad2), RE, *_fold_first_layer(
        p['om_edge_enc'], p['om_mean'], p['om_std']))
    e_mo = _encode(jnp.pad(mo_attr, pad2), RE, *_fold_first_layer(
        p['mo_edge_enc'], p['mo_mean'], p['mo_std']))

    gidx_om = jnp.stack([_widx(om_index[0]), _widx(om_index[1])], axis=1)
    gidx_mo = jnp.stack([_widx(mo_index[0]), _widx(mo_index[1])], axis=1)
    sidx_om = _widx(om_index[1])
    sidx_mo = _widx(mo_index[1])
    zeros = jnp.zeros((NM, LAT), jnp.float32)
    sc_gather, sc_scatter = _sc_kernels()

    for st in range(STEPS):
        sp = p['steps'][st]
        om_w1, om_b1 = sp['om_e'][0]
        mo_w1, mo_b1 = sp['mo_e'][0]
        t_om_src, t_mo_dst = _proj(ns_o, om_w1[LAT:2 * LAT], mo_w1[2 * LAT:])
        t_om_dst, t_mo_src = _proj(ns_m, om_w1[2 * LAT:], mo_w1[LAT:2 * LAT])

        g_om_s, g_om_d = sc_gather(t_om_src, t_om_dst, gidx_om)
        g_mo_s, g_mo_d = sc_gather(t_mo_src, t_mo_dst, gidx_mo)

        e_om = _tail(e_om, g_om_s, g_om_d, om_w1[:LAT], om_b1[None],
                     *_row_biases(sp['om_e'][1:]))
        e_mo = _tail(e_mo, g_mo_s, g_mo_d, mo_w1[:LAT], mo_b1[None],
                     *_row_biases(sp['mo_e'][1:]))

        agg_m2 = sc_scatter(e_om, sidx_om, zeros).reshape(2, NM, LAT)
        agg_o2 = sc_scatter(e_mo, sidx_mo, zeros).reshape(2, NM, LAT)

        me_w1, me_b1 = sp['mesh_n'][0]
        ob_w1, ob_b1 = sp['obj_n'][0]
        ns_m = _nupd(ns_m, agg_m2, me_w1[:LAT], me_w1[LAT:], me_b1[None],
                     *_row_biases(sp['mesh_n'][1:]))
        ns_o = _nupd(ns_o, agg_o2, ob_w1[:LAT], ob_w1[LAT:], ob_b1[None],
                     *_row_biases(sp['obj_n'][1:]))

    m_acc = _dec(ns_m, *_row_biases(p['mesh_dec']))
    o_acc = _dec(ns_o, *_row_biases(p['obj_dec']))
    return m_acc, o_acc


# speedup vs baseline: 1.2900x; 1.0107x over previous
import functools

import jax
import jax.numpy as jnp
from jax import lax
from jax.experimental import pallas as pl
from jax.experimental.pallas import tpu as pltpu
from jax.experimental.pallas import tpu_sc as plsc

NM = 10000
E = 160000
EP = 163840
LAT = 64
NC = 2
NSUB = 16
NW = NC * NSUB
EPW = EP // NW
CHUNK = 128
NCHTOT = EPW // CHUNK
GBLK = 128
GNB = EPW // GBLK
SBLK = 256
SNB = EPW // SBLK
SNCH = SBLK // CHUNK
NPS = NM // NSUB
RN = 2000
RE = 2048
STEPS = 3
EPS = 1e-5


def _ln(y):
    m = jnp.mean(y, axis=-1, keepdims=True)
    d = y - m
    v = jnp.mean(d * d, axis=-1, keepdims=True)
    return d * lax.rsqrt(v + EPS)


def _enc_body(x_ref, w1, b1, w2, b2, w3, b3, o_ref):
    x = x_ref[...]
    h = jnp.maximum(x @ w1[...] + b1[...], 0.0)
    h = jnp.maximum(h @ w2[...] + b2[...], 0.0)
    y = h @ w3[...] + b3[...]
    o_ref[...] = _ln(y)


def _full(shape):
    return pl.BlockSpec(shape, lambda b: (0,) * len(shape))


def _encode(x, rows, w1, b1, w2, b2, w3, b3):
    n, d = x.shape
    return pl.pallas_call(
        _enc_body,
        grid=(n // rows,),
        in_specs=[
            pl.BlockSpec((rows, d), lambda b: (b, 0)),
            _full((d, LAT)), _full((1, LAT)),
            _full((LAT, LAT)), _full((1, LAT)),
            _full((LAT, LAT)), _full((1, LAT)),
        ],
        out_specs=pl.BlockSpec((rows, LAT), lambda b: (b, 0)),
        out_shape=jax.ShapeDtypeStruct((n, LAT), jnp.float32),
    )(x, w1, b1, w2, b2, w3, b3)


def _encp_body(x_ref, w1, b1, w2, b2, w3, b3, wa, wb, o_ref, oa_ref, ob_ref):
    x = x_ref[...]
    h = jnp.maximum(x @ w1[...] + b1[...], 0.0)
    h = jnp.maximum(h @ w2[...] + b2[...], 0.0)
    y = _ln(h @ w3[...] + b3[...])
    o_ref[...] = y
    oa_ref[...] = y @ wa[...]
    ob_ref[...] = y @ wb[...]


def _encode_proj(x, w1, b1, w2, b2, w3, b3, wa, wb):
    n, d = x.shape
    spec = pl.BlockSpec((RN, LAT), lambda b: (b, 0))
    return pl.pallas_call(
        _encp_body,
        grid=(n // RN,),
        in_specs=[
            pl.BlockSpec((RN, d), lambda b: (b, 0)),
            _full((d, LAT)), _full((1, LAT)),
            _full((LAT, LAT)), _full((1, LAT)),
            _full((LAT, LAT)), _full((1, LAT)),
            _full((LAT, LAT)), _full((LAT, LAT)),
        ],
        out_specs=[spec, spec, spec],
        out_shape=[jax.ShapeDtypeStruct((n, LAT), jnp.float32)] * 3,
    )(x, w1, b1, w2, b2, w3, b3, wa, wb)


def _tail_body(e_ref, gs_ref, gd_ref, w1, b1, w2, b2, w3, b3, o_ref):
    e = e_ref[...]
    x = e @ w1[...] + gs_ref[...] + gd_ref[...] + b1[...]
    h = jnp.maximum(x, 0.0)
    h = jnp.maximum(h @ w2[...] + b2[...], 0.0)
    y = h @ w3[...] + b3[...]
    out = e + _ln(y)
    rows = pl.program_id(0) * RE + lax.broadcasted_iota(jnp.int32, (RE, 1), 0)
    o_ref[...] = jnp.where(rows < E, out, 0.0)


def _tail(e, gs, gd, w1, b1, w2, b2, w3, b3):
    espec = pl.BlockSpec((RE, LAT), lambda b: (b, 0))
    return pl.pallas_call(
        _tail_body,
        grid=(EP // RE,),
        in_specs=[espec, espec, espec,
                  _full((LAT, LAT)), _full((1, LAT)),
                  _full((LAT, LAT)), _full((1, LAT)),
                  _full((LAT, LAT)), _full((1, LAT))],
        out_specs=espec,
        out_shape=jax.ShapeDtypeStruct((EP, LAT), jnp.float32),
    )(e, gs, gd, w1, b1, w2, b2, w3, b3)


def _nupd_body(n_ref, a_ref, w1a, w1b, b1, w2, b2, w3, b3, o_ref):
    x = n_ref[...]
    agg = a_ref[0] + a_ref[1]
    h = jnp.maximum(x @ w1a[...] + agg @ w1b[...] + b1[...], 0.0)
    h = jnp.maximum(h @ w2[...] + b2[...], 0.0)
    y = h @ w3[...] + b3[...]
    o_ref[...] = x + _ln(y)


def _nupd(ns, agg2, w1a, w1b, b1, w2, b2, w3, b3):
    spec = pl.BlockSpec((RN, LAT), lambda b: (b, 0))
    return pl.pallas_call(
        _nupd_body,
        grid=(NM // RN,),
        in_specs=[spec,
                  pl.BlockSpec((2, RN, LAT), lambda b: (0, b, 0)),
                  _full((LAT, LAT)), _full((LAT, LAT)), _full((1, LAT)),
                  _full((LAT, LAT)), _full((1, LAT)),
                  _full((LAT, LAT)), _full((1, LAT))],
        out_specs=spec,
        out_shape=jax.ShapeDtypeStruct((NM, LAT), jnp.float32),
    )(ns, agg2, w1a, w1b, b1, w2, b2, w3, b3)


def _nupdp_body(n_ref, a_ref, w1a, w1b, b1, w2, b2, w3, b3, wa, wb,
                o_ref, oa_ref, ob_ref):
    x = n_ref[...]
    agg = a_ref[0] + a_ref[1]
    h = jnp.maximum(x @ w1a[...] + agg @ w1b[...] + b1[...], 0.0)
    h = jnp.maximum(h @ w2[...] + b2[...], 0.0)
    y = h @ w3[...] + b3[...]
    out = x + _ln(y)
    o_ref[...] = out
    oa_ref[...] = out @ wa[...]
    ob_ref[...] = out @ wb[...]


def _nupd_proj(ns, agg2, w1a, w1b, b1, w2, b2, w3, b3, wa, wb):
    spec = pl.BlockSpec((RN, LAT), lambda b: (b, 0))
    return pl.pallas_call(
        _nupdp_body,
        grid=(NM // RN,),
        in_specs=[spec,
                  pl.BlockSpec((2, RN, LAT), lambda b: (0, b, 0)),
                  _full((LAT, LAT)), _full((LAT, LAT)), _full((1, LAT)),
                  _full((LAT, LAT)), _full((1, LAT)),
                  _full((LAT, LAT)), _full((1, LAT)),
                  _full((LAT, LAT)), _full((LAT, LAT))],
        out_specs=[spec, spec, spec],
        out_shape=[jax.ShapeDtypeStruct((NM, LAT), jnp.float32)] * 3,
    )(ns, agg2, w1a, w1b, b1, w2, b2, w3, b3, wa, wb)


def _dec_body(n_ref, w1, b1, w2, b2, w3, b3, o_ref):
    x = n_ref[...]
    h = jnp.maximum(x @ w1[...] + b1[...], 0.0)
    h = jnp.maximum(h @ w2[...] + b2[...], 0.0)
    o_ref[...] = h @ w3[...] + b3[...]


def _dec(ns, w1, b1, w2, b2, w3, b3):
    return pl.pallas_call(
        _dec_body,
        grid=(NM // RN,),
        in_specs=[pl.BlockSpec((RN, LAT), lambda b: (b, 0)),
                  _full((LAT, LAT)), _full((1, LAT)),
                  _full((LAT, LAT)), _full((1, LAT)),
                  _full((LAT, 3)), _full((1, 3))],
        out_specs=pl.BlockSpec((RN, 3), lambda b: (b, 0)),
        out_shape=jax.ShapeDtypeStruct((NM, 3), jnp.float32),
    )(ns, w1, b1, w2, b2, w3, b3)


@functools.cache
def _sc_kernels():
    mesh = plsc.VectorSubcoreMesh(core_axis_name="c", subcore_axis_name="s",
                                  num_cores=NC, num_subcores=NSUB)

    @functools.partial(
        pl.kernel,
        out_type=(
            jax.ShapeDtypeStruct((EP, LAT), jnp.float32),
            jax.ShapeDtypeStruct((EP, LAT), jnp.float32),
        ),
        mesh=mesh,
        scratch_types=[
            pltpu.VMEM((2, NCHTOT, CHUNK), jnp.int32),
            pltpu.VMEM((GBLK, LAT), jnp.float32),
            pltpu.VMEM((GBLK, LAT), jnp.float32),
            pltpu.VMEM((GBLK, LAT), jnp.float32),
            pltpu.VMEM((GBLK, LAT), jnp.float32),
            pltpu.VMEM_SHARED((NM, LAT), jnp.float32),
            pltpu.VMEM_SHARED((NM, LAT), jnp.float32),
            pltpu.SemaphoreType.DMA,
            pltpu.SemaphoreType.DMA,
        ],
        compiler_params=pltpu.CompilerParams(use_tc_tiling_on_sc=False),
    )
    def _sc_gather(tsrc, tdst, gidx, gs_out, gd_out,
                   idx_v, ba0, bb0, ba1, bb1, spms, spmd, sem0, sem1):
        c = lax.axis_index("c")
        s = lax.axis_index("s")
        wid = s * NC + c
        pltpu.sync_copy(gidx.at[wid], idx_v)
        off = s * NPS
        pltpu.sync_copy(tsrc.at[pl.ds(off, NPS)], spms.at[pl.ds(off, NPS)])
        pltpu.sync_copy(tdst.at[pl.ds(off, NPS)], spmd.at[pl.ds(off, NPS)])
        plsc.subcore_barrier()
        bufs = ((ba0, bb0, sem0), (ba1, bb1, sem1))

        def descs(b, par):
            ba, bb_, sem = bufs[par]
            return [
                pltpu.make_async_copy(spms.at[idx_v.at[0, b]], ba, sem),
                pltpu.make_async_copy(spmd.at[idx_v.at[1, b]], bb_, sem),
            ]

        def g_start(b, par):
            for d in descs(b, par):
                d.start()

        def g_wait(b, par):
            for d in descs(b, par):
                d.wait()

        def wr(b, par):
            ba, bb_, _ = bufs[par]
            base = wid * EPW + b * GBLK
            pltpu.sync_copy(ba, gs_out.at[pl.ds(base, GBLK)])
            pltpu.sync_copy(bb_, gd_out.at[pl.ds(base, GBLK)])

        g_start(0, 0)

        def body(bb, carry):
            b0 = 2 * bb
            g_start(b0 + 1, 1)
            g_wait(b0, 0)
            wr(b0, 0)
            pl.when(bb < GNB // 2 - 1)(lambda: g_start(b0 + 2, 0))
            g_wait(b0 + 1, 1)
            wr(b0 + 1, 1)
            return carry

        lax.fori_loop(0, GNB // 2, body, 0)

    @functools.partial(
        pl.kernel,
        out_type=jax.ShapeDtypeStruct((2 * NM, LAT), jnp.float32),
        mesh=mesh,
        scratch_types=[
            pltpu.VMEM((NCHTOT, CHUNK), jnp.int32),
            pltpu.VMEM((SBLK, LAT), jnp.float32),
            pltpu.VMEM((SBLK, LAT), jnp.float32),
            pltpu.VMEM((SBLK, LAT), jnp.float32),
            pltpu.VMEM((SBLK, LAT), jnp.float32),
            pltpu.VMEM_SHARED((NM, LAT), jnp.float32),
            pltpu.SemaphoreType.DMA,
            pltpu.SemaphoreType.DMA,
            pltpu.SemaphoreType.DMA,
            pltpu.SemaphoreType.DMA,
            pltpu.SemaphoreType.DMA,
            pltpu.SemaphoreType.DMA,
            pltpu.SemaphoreType.DMA,
            pltpu.SemaphoreType.DMA,
        ],
        compiler_params=pltpu.CompilerParams(use_tc_tiling_on_sc=False),
    )
    def _sc_scatter(enew, sidx, zeros, agg_out, idx_v, r0, r1, r2, r3,
                    spm, l0, l1, l2, l3, a0, a1, a2, a3):
        c = lax.axis_index("c")
        s = lax.axis_index("s")
        wid = s * NC + c
        rows = (r0, r1, r2, r3)
        lsems = (l0, l1, l2, l3)
        asems = (a0, a1, a2, a3)
        pltpu.sync_copy(zeros.at[pl.ds(s * NPS, NPS)],
                        spm.at[pl.ds(s * NPS, NPS)])
        pltpu.sync_copy(sidx.at[wid], idx_v)
        plsc.subcore_barrier()

        def load_desc(k, b):
            base = wid * EPW + b * SBLK
            return pltpu.make_async_copy(
                enew.at[pl.ds(base, SBLK)], rows[k], lsems[k])

        def add_descs(k, b):
            out = []
            for cc in range(SNCH):
                j = b * SNCH + cc
                out.append(pltpu.make_async_copy(
                    rows[k].at[pl.ds(cc * CHUNK, CHUNK)],
                    spm.at[idx_v.at[j]], asems[k]))
            return out

        for k in range(4):
            load_desc(k, k).start()

        def body(bb, carry):
            for k in range(4):
                b = 4 * bb + k
                load_desc(k, b).wait()
                for d in add_descs(k, b):
                    d.start(add=True)
            for k in range(4):
                b = 4 * bb + k

                def refill(k=k, b=b):
                    for d in add_descs(k, b):
                        d.wait()
                    load_desc(k, b + 4).start()

                pl.when(bb < SNB // 4 - 1)(refill)
            return carry

        lax.fori_loop(0, SNB // 4, body, 0)
        for k in range(4):
            for d in add_descs(k, SNB - 4 + k):
                d.wait()
        plsc.subcore_barrier()
        off = s * NPS
        pltpu.sync_copy(spm.at[pl.ds(off, NPS)],
                        agg_out.at[pl.ds(c * NM + off, NPS)])

    return _sc_gather, _sc_scatter


def _fold_first_layer(ps, mean, std):
    (w1, b1), (w2, b2), (w3, b3) = ps
    w1f = w1 / std[:, None]
    b1f = b1 - (mean / std) @ w1
    return w1f, b1f[None], w2, b2[None], w3, b3[None]


def _row_biases(ps):
    out = []
    for w, b in ps:
        out.append(w)
        out.append(b[None])
    return out


def _widx(v):
    return jnp.pad(v.astype(jnp.int32), (0, EP - E)).reshape(
        NW, NCHTOT, CHUNK)


def kernel(mesh_position, mesh_properties, mesh_kinematic, obj_position,
           obj_properties, obj_kinematic, om_index, om_attr, mo_index,
           mo_attr, params):
    p = params
    m_x = jnp.concatenate([
        mesh_position[1] - mesh_position[0],
        mesh_position[2] - mesh_position[1],
        mesh_properties, mesh_kinematic], axis=-1)
    o_x = jnp.concatenate([
        obj_position[1] - obj_position[0],
        obj_position[2] - obj_position[1],
        obj_properties, obj_kinematic], axis=-1)

    def _pw(sp):
        om_w1 = sp['om_e'][0][0]
        mo_w1 = sp['mo_e'][0][0]
        return ((om_w1[LAT:2 * LAT], mo_w1[2 * LAT:]),
                (om_w1[2 * LAT:], mo_w1[LAT:2 * LAT]))

    pw0 = _pw(p['steps'][0])
    ns_o, t_om_src, t_mo_dst = _encode_proj(*(
        (o_x,) + _fold_first_layer(p['obj_enc'], p['node_mean'],
                                   p['node_std']) + pw0[0]))
    ns_m, t_om_dst, t_mo_src = _encode_proj(*(
        (m_x,) + _fold_first_layer(p['mesh_enc'], p['node_mean'],
                                   p['node_std']) + pw0[1]))

    pad2 = ((0, EP - E), (0, 0))
    e_om = _encode(jnp.pad(om_attr, pad2), RE, *_fold_first_layer(
        p['om_edge_enc'], p['om_mean'], p['om_std']))
    e_mo = _encode(jnp.pad(mo_attr, pad2), RE, *_fold_first_layer(
        p['mo_edge_enc'], p['mo_mean'], p['mo_std']))

    gidx_om = jnp.stack([_widx(om_index[0]), _widx(om_index[1])], axis=1)
    gidx_mo = jnp.stack([_widx(mo_index[0]), _widx(mo_index[1])], axis=1)
    sidx_om = _widx(om_index[1])
    sidx_mo = _widx(mo_index[1])
    zeros = jnp.zeros((NM, LAT), jnp.float32)
    sc_gather, sc_scatter = _sc_kernels()

    for st in range(STEPS):
        sp = p['steps'][st]
        om_w1, om_b1 = sp['om_e'][0]
        mo_w1, mo_b1 = sp['mo_e'][0]

        g_om_s, g_om_d = sc_gather(t_om_src, t_om_dst, gidx_om)
        g_mo_s, g_mo_d = sc_gather(t_mo_src, t_mo_dst, gidx_mo)

        e_om = _tail(e_om, g_om_s, g_om_d, om_w1[:LAT], om_b1[None],
                     *_row_biases(sp['om_e'][1:]))
        e_mo = _tail(e_mo, g_mo_s, g_mo_d, mo_w1[:LAT], mo_b1[None],
                     *_row_biases(sp['mo_e'][1:]))

        agg_m2 = sc_scatter(e_om, sidx_om, zeros).reshape(2, NM, LAT)
        agg_o2 = sc_scatter(e_mo, sidx_mo, zeros).reshape(2, NM, LAT)

        me_w1, me_b1 = sp['mesh_n'][0]
        ob_w1, ob_b1 = sp['obj_n'][0]
        m_args = (agg_m2, me_w1[:LAT], me_w1[LAT:], me_b1[None],
                  *_row_biases(sp['mesh_n'][1:]))
        o_args = (agg_o2, ob_w1[:LAT], ob_w1[LAT:], ob_b1[None],
                  *_row_biases(sp['obj_n'][1:]))
        if st < STEPS - 1:
            pwn = _pw(p['steps'][st + 1])
            ns_m, t_om_dst, t_mo_src = _nupd_proj(ns_m, *m_args, *pwn[1])
            ns_o, t_om_src, t_mo_dst = _nupd_proj(ns_o, *o_args, *pwn[0])
        else:
            ns_m = _nupd(ns_m, *m_args)
            ns_o = _nupd(ns_o, *o_args)

    m_acc = _dec(ns_m, *_row_biases(p['mesh_dec']))
    o_acc = _dec(ns_o, *_row_biases(p['obj_dec']))
    return m_acc, o_acc


# speedup vs baseline: 1.4602x; 1.1319x over previous
import functools

import jax
import jax.numpy as jnp
from jax import lax
from jax.experimental import pallas as pl
from jax.experimental.pallas import tpu as pltpu
from jax.experimental.pallas import tpu_sc as plsc

NM = 10000
E = 160000
EP = 163840
LAT = 64
NC = 2
NSUB = 16
NW = NC * NSUB
EPW = EP // NW
CHUNK = 128
NCHTOT = EPW // CHUNK
GBLK = 128
GNB = EPW // GBLK
SBLK = 256
SNB = EPW // SBLK
SNCH = SBLK // CHUNK
NPS = NM // NSUB
RN = 2000
RE = 4096
STEPS = 3
EPS = 1e-5


def _ln(y):
    m = jnp.mean(y, axis=-1, keepdims=True)
    d = y - m
    v = jnp.mean(d * d, axis=-1, keepdims=True)
    return d * lax.rsqrt(v + EPS)


def _enc_body(x_ref, w1, b1, w2, b2, w3, b3, o_ref):
    x = x_ref[...]
    h = jnp.maximum(x @ w1[...] + b1[...], 0.0)
    h = jnp.maximum(h @ w2[...] + b2[...], 0.0)
    y = h @ w3[...] + b3[...]
    o_ref[...] = _ln(y)


def _full(shape):
    return pl.BlockSpec(shape, lambda b: (0,) * len(shape))


def _encode(x, rows, w1, b1, w2, b2, w3, b3):
    n, d = x.shape
    return pl.pallas_call(
        _enc_body,
        grid=(n // rows,),
        in_specs=[
            pl.BlockSpec((rows, d), lambda b: (b, 0)),
            _full((d, LAT)), _full((1, LAT)),
            _full((LAT, LAT)), _full((1, LAT)),
            _full((LAT, LAT)), _full((1, LAT)),
        ],
        out_specs=pl.BlockSpec((rows, LAT), lambda b: (b, 0)),
        out_shape=jax.ShapeDtypeStruct((n, LAT), jnp.float32),
    )(x, w1, b1, w2, b2, w3, b3)


def _encp_body(x_ref, w1, b1, w2, b2, w3, b3, wa, wb, o_ref, oa_ref, ob_ref):
    x = x_ref[...]
    h = jnp.maximum(x @ w1[...] + b1[...], 0.0)
    h = jnp.maximum(h @ w2[...] + b2[...], 0.0)
    y = _ln(h @ w3[...] + b3[...])
    o_ref[...] = y
    oa_ref[...] = y @ wa[...]
    ob_ref[...] = y @ wb[...]


def _encode_proj(x, w1, b1, w2, b2, w3, b3, wa, wb):
    n, d = x.shape
    spec = pl.BlockSpec((RN, LAT), lambda b: (b, 0))
    return pl.pallas_call(
        _encp_body,
        grid=(n // RN,),
        in_specs=[
            pl.BlockSpec((RN, d), lambda b: (b, 0)),
            _full((d, LAT)), _full((1, LAT)),
            _full((LAT, LAT)), _full((1, LAT)),
            _full((LAT, LAT)), _full((1, LAT)),
            _full((LAT, LAT)), _full((LAT, LAT)),
        ],
        out_specs=[spec, spec, spec],
        out_shape=[jax.ShapeDtypeStruct((n, LAT), jnp.float32)] * 3,
    )(x, w1, b1, w2, b2, w3, b3, wa, wb)


def _tail_body(e_ref, gs_ref, gd_ref, w1, b1, w2, b2, w3, b3, o_ref):
    e = e_ref[...]
    x = e @ w1[...] + gs_ref[...] + gd_ref[...] + b1[...]
    h = jnp.maximum(x, 0.0)
    h = jnp.maximum(h @ w2[...] + b2[...], 0.0)
    y = h @ w3[...] + b3[...]
    out = e + _ln(y)
    rows = pl.program_id(0) * RE + lax.broadcasted_iota(jnp.int32, (RE, 1), 0)
    o_ref[...] = jnp.where(rows < E, out, 0.0)


def _tail(e, gs, gd, w1, b1, w2, b2, w3, b3):
    espec = pl.BlockSpec((RE, LAT), lambda b: (b, 0))
    return pl.pallas_call(
        _tail_body,
        grid=(EP // RE,),
        in_specs=[espec, espec, espec,
                  _full((LAT, LAT)), _full((1, LAT)),
                  _full((LAT, LAT)), _full((1, LAT)),
                  _full((LAT, LAT)), _full((1, LAT))],
        out_specs=espec,
        out_shape=jax.ShapeDtypeStruct((EP, LAT), jnp.float32),
    )(e, gs, gd, w1, b1, w2, b2, w3, b3)


def _tail0_body(ea_ref, gs_ref, gd_ref, ew1, eb1, ew2, eb2, ew3, eb3,
                w1, b1, w2, b2, w3, b3, o_ref):
    a = ea_ref[...]
    h = jnp.maximum(a @ ew1[...] + eb1[...], 0.0)
    h = jnp.maximum(h @ ew2[...] + eb2[...], 0.0)
    e = _ln(h @ ew3[...] + eb3[...])
    x = e @ w1[...] + gs_ref[...] + gd_ref[...] + b1[...]
    h = jnp.maximum(x, 0.0)
    h = jnp.maximum(h @ w2[...] + b2[...], 0.0)
    y = h @ w3[...] + b3[...]
    out = e + _ln(y)
    rows = pl.program_id(0) * RE + lax.broadcasted_iota(jnp.int32, (RE, 1), 0)
    o_ref[...] = jnp.where(rows < E, out, 0.0)


def _tail0(ea, gs, gd, ew, w1, b1, w2, b2, w3, b3):
    espec = pl.BlockSpec((RE, LAT), lambda b: (b, 0))
    return pl.pallas_call(
        _tail0_body,
        grid=(EP // RE,),
        in_specs=[pl.BlockSpec((RE, 8), lambda b: (b, 0)), espec, espec,
                  _full((8, LAT)), _full((1, LAT)),
                  _full((LAT, LAT)), _full((1, LAT)),
                  _full((LAT, LAT)), _full((1, LAT)),
                  _full((LAT, LAT)), _full((1, LAT)),
                  _full((LAT, LAT)), _full((1, LAT)),
                  _full((LAT, LAT)), _full((1, LAT))],
        out_specs=espec,
        out_shape=jax.ShapeDtypeStruct((EP, LAT), jnp.float32),
    )(ea, gs, gd, *ew, w1, b1, w2, b2, w3, b3)


def _nupd_body(n_ref, a_ref, w1a, w1b, b1, w2, b2, w3, b3, o_ref):
    x = n_ref[...]
    agg = a_ref[0] + a_ref[1]
    h = jnp.maximum(x @ w1a[...] + agg @ w1b[...] + b1[...], 0.0)
    h = jnp.maximum(h @ w2[...] + b2[...], 0.0)
    y = h @ w3[...] + b3[...]
    o_ref[...] = x + _ln(y)


def _nupd(ns, agg2, w1a, w1b, b1, w2, b2, w3, b3):
    spec = pl.BlockSpec((RN, LAT), lambda b: (b, 0))
    return pl.pallas_call(
        _nupd_body,
        grid=(NM // RN,),
        in_specs=[spec,
                  pl.BlockSpec((2, RN, LAT), lambda b: (0, b, 0)),
                  _full((LAT, LAT)), _full((LAT, LAT)), _full((1, LAT)),
                  _full((LAT, LAT)), _full((1, LAT)),
                  _full((LAT, LAT)), _full((1, LAT))],
        out_specs=spec,
        out_shape=jax.ShapeDtypeStruct((NM, LAT), jnp.float32),
    )(ns, agg2, w1a, w1b, b1, w2, b2, w3, b3)


def _nupdp_body(n_ref, a_ref, w1a, w1b, b1, w2, b2, w3, b3, wa, wb,
                o_ref, oa_ref, ob_ref):
    x = n_ref[...]
    agg = a_ref[0] + a_ref[1]
    h = jnp.maximum(x @ w1a[...] + agg @ w1b[...] + b1[...], 0.0)
    h = jnp.maximum(h @ w2[...] + b2[...], 0.0)
    y = h @ w3[...] + b3[...]
    out = x + _ln(y)
    o_ref[...] = out
    oa_ref[...] = out @ wa[...]
    ob_ref[...] = out @ wb[...]


def _nupd_proj(ns, agg2, w1a, w1b, b1, w2, b2, w3, b3, wa, wb):
    spec = pl.BlockSpec((RN, LAT), lambda b: (b, 0))
    return pl.pallas_call(
        _nupdp_body,
        grid=(NM // RN,),
        in_specs=[spec,
                  pl.BlockSpec((2, RN, LAT), lambda b: (0, b, 0)),
                  _full((LAT, LAT)), _full((LAT, LAT)), _full((1, LAT)),
                  _full((LAT, LAT)), _full((1, LAT)),
                  _full((LAT, LAT)), _full((1, LAT)),
                  _full((LAT, LAT)), _full((LAT, LAT))],
        out_specs=[spec, spec, spec],
        out_shape=[jax.ShapeDtypeStruct((NM, LAT), jnp.float32)] * 3,
    )(ns, agg2, w1a, w1b, b1, w2, b2, w3, b3, wa, wb)


def _nupdd_body(n_ref, a_ref, w1a, w1b, b1, w2, b2, w3, b3,
                d1, c1, d2, c2, d3, c3, o_ref):
    x = n_ref[...]
    agg = a_ref[0] + a_ref[1]
    h = jnp.maximum(x @ w1a[...] + agg @ w1b[...] + b1[...], 0.0)
    h = jnp.maximum(h @ w2[...] + b2[...], 0.0)
    y = h @ w3[...] + b3[...]
    n = x + _ln(y)
    h = jnp.maximum(n @ d1[...] + c1[...], 0.0)
    h = jnp.maximum(h @ d2[...] + c2[...], 0.0)
    o_ref[...] = h @ d3[...] + c3[...]


def _nupd_dec(ns, agg2, w1a, w1b, b1, w2, b2, w3, b3, dw):
    return pl.pallas_call(
        _nupdd_body,
        grid=(NM // RN,),
        in_specs=[pl.BlockSpec((RN, LAT), lambda b: (b, 0)),
                  pl.BlockSpec((2, RN, LAT), lambda b: (0, b, 0)),
                  _full((LAT, LAT)), _full((LAT, LAT)), _full((1, LAT)),
                  _full((LAT, LAT)), _full((1, LAT)),
                  _full((LAT, LAT)), _full((1, LAT)),
                  _full((LAT, LAT)), _full((1, LAT)),
                  _full((LAT, LAT)), _full((1, LAT)),
                  _full((LAT, 3)), _full((1, 3))],
        out_specs=pl.BlockSpec((RN, 3), lambda b: (b, 0)),
        out_shape=jax.ShapeDtypeStruct((NM, 3), jnp.float32),
    )(ns, agg2, w1a, w1b, b1, w2, b2, w3, b3, *dw)


def _dec_body(n_ref, w1, b1, w2, b2, w3, b3, o_ref):
    x = n_ref[...]
    h = jnp.maximum(x @ w1[...] + b1[...], 0.0)
    h = jnp.maximum(h @ w2[...] + b2[...], 0.0)
    o_ref[...] = h @ w3[...] + b3[...]


def _dec(ns, w1, b1, w2, b2, w3, b3):
    return pl.pallas_call(
        _dec_body,
        grid=(NM // RN,),
        in_specs=[pl.BlockSpec((RN, LAT), lambda b: (b, 0)),
                  _full((LAT, LAT)), _full((1, LAT)),
                  _full((LAT, LAT)), _full((1, LAT)),
                  _full((LAT, 3)), _full((1, 3))],
        out_specs=pl.BlockSpec((RN, 3), lambda b: (b, 0)),
        out_shape=jax.ShapeDtypeStruct((NM, 3), jnp.float32),
    )(ns, w1, b1, w2, b2, w3, b3)


@functools.cache
def _sc_kernels():
    mesh = plsc.VectorSubcoreMesh(core_axis_name="c", subcore_axis_name="s",
                                  num_cores=NC, num_subcores=NSUB)

    @functools.partial(
        pl.kernel,
        out_type=(
            jax.ShapeDtypeStruct((EP, LAT), jnp.float32),
            jax.ShapeDtypeStruct((EP, LAT), jnp.float32),
        ),
        mesh=mesh,
        scratch_types=[
            pltpu.VMEM((2, NCHTOT, CHUNK), jnp.int32),
            pltpu.VMEM((GBLK, LAT), jnp.float32),
            pltpu.VMEM((GBLK, LAT), jnp.float32),
            pltpu.VMEM((GBLK, LAT), jnp.float32),
            pltpu.VMEM((GBLK, LAT), jnp.float32),
            pltpu.VMEM_SHARED((NM, LAT), jnp.float32),
            pltpu.VMEM_SHARED((NM, LAT), jnp.float32),
            pltpu.SemaphoreType.DMA,
            pltpu.SemaphoreType.DMA,
        ],
        compiler_params=pltpu.CompilerParams(use_tc_tiling_on_sc=False),
    )
    def _sc_gather(tsrc, tdst, gidx, gs_out, gd_out,
                   idx_v, ba0, bb0, ba1, bb1, spms, spmd, sem0, sem1):
        c = lax.axis_index("c")
        s = lax.axis_index("s")
        wid = s * NC + c
        pltpu.sync_copy(gidx.at[wid], idx_v)
        off = s * NPS
        pltpu.sync_copy(tsrc.at[pl.ds(off, NPS)], spms.at[pl.ds(off, NPS)])
        pltpu.sync_copy(tdst.at[pl.ds(off, NPS)], spmd.at[pl.ds(off, NPS)])
        plsc.subcore_barrier()
        bufs = ((ba0, bb0, sem0), (ba1, bb1, sem1))

        def descs(b, par):
            ba, bb_, sem = bufs[par]
            return [
                pltpu.make_async_copy(spms.at[idx_v.at[0, b]], ba, sem),
                pltpu.make_async_copy(spmd.at[idx_v.at[1, b]], bb_, sem),
            ]

        def g_start(b, par):
            for d in descs(b, par):
                d.start()

        def g_wait(b, par):
            for d in descs(b, par):
                d.wait()

        def wr(b, par):
            ba, bb_, _ = bufs[par]
            base = wid * EPW + b * GBLK
            pltpu.sync_copy(ba, gs_out.at[pl.ds(base, GBLK)])
            pltpu.sync_copy(bb_, gd_out.at[pl.ds(base, GBLK)])

        g_start(0, 0)

        def body(bb, carry):
            b0 = 2 * bb
            g_start(b0 + 1, 1)
            g_wait(b0, 0)
            wr(b0, 0)
            pl.when(bb < GNB // 2 - 1)(lambda: g_start(b0 + 2, 0))
            g_wait(b0 + 1, 1)
            wr(b0 + 1, 1)
            return carry

        lax.fori_loop(0, GNB // 2, body, 0)

    @functools.partial(
        pl.kernel,
        out_type=jax.ShapeDtypeStruct((2 * NM, LAT), jnp.float32),
        mesh=mesh,
        scratch_types=[
            pltpu.VMEM((NCHTOT, CHUNK), jnp.int32),
            pltpu.VMEM((SBLK, LAT), jnp.float32),
            pltpu.VMEM((SBLK, LAT), jnp.float32),
            pltpu.VMEM((SBLK, LAT), jnp.float32),
            pltpu.VMEM((SBLK, LAT), jnp.float32),
            pltpu.VMEM_SHARED((NM, LAT), jnp.float32),
            pltpu.SemaphoreType.DMA,
            pltpu.SemaphoreType.DMA,
            pltpu.SemaphoreType.DMA,
            pltpu.SemaphoreType.DMA,
            pltpu.SemaphoreType.DMA,
            pltpu.SemaphoreType.DMA,
            pltpu.SemaphoreType.DMA,
            pltpu.SemaphoreType.DMA,
        ],
        compiler_params=pltpu.CompilerParams(use_tc_tiling_on_sc=False),
    )
    def _sc_scatter(enew, sidx, zeros, agg_out, idx_v, r0, r1, r2, r3,
                    spm, l0, l1, l2, l3, a0, a1, a2, a3):
        c = lax.axis_index("c")
        s = lax.axis_index("s")
        wid = s * NC + c
        rows = (r0, r1, r2, r3)
        lsems = (l0, l1, l2, l3)
        asems = (a0, a1, a2, a3)
        pltpu.sync_copy(zeros.at[pl.ds(s * NPS, NPS)],
                        spm.at[pl.ds(s * NPS, NPS)])
        pltpu.sync_copy(sidx.at[wid], idx_v)
        plsc.subcore_barrier()

        def load_desc(k, b):
            base = wid * EPW + b * SBLK
            return pltpu.make_async_copy(
                enew.at[pl.ds(base, SBLK)], rows[k], lsems[k])

        def add_descs(k, b):
            out = []
            for cc in range(SNCH):
                j = b * SNCH + cc
                out.append(pltpu.make_async_copy(
                    rows[k].at[pl.ds(cc * CHUNK, CHUNK)],
                    spm.at[idx_v.at[j]], asems[k]))
            return out

        for k in range(4):
            load_desc(k, k).start()

        def body(bb, carry):
            for k in range(4):
                b = 4 * bb + k
                load_desc(k, b).wait()
                for d in add_descs(k, b):
                    d.start(add=True)
            for k in range(4):
                b = 4 * bb + k

                def refill(k=k, b=b):
                    for d in add_descs(k, b):
                        d.wait()
                    load_desc(k, b + 4).start()

                pl.when(bb < SNB // 4 - 1)(refill)
            return carry

        lax.fori_loop(0, SNB // 4, body, 0)
        for k in range(4):
            for d in add_descs(k, SNB - 4 + k):
                d.wait()
        plsc.subcore_barrier()
        off = s * NPS
        pltpu.sync_copy(spm.at[pl.ds(off, NPS)],
                        agg_out.at[pl.ds(c * NM + off, NPS)])

    return _sc_gather, _sc_scatter


def _fold_first_layer(ps, mean, std):
    (w1, b1), (w2, b2), (w3, b3) = ps
    w1f = w1 / std[:, None]
    b1f = b1 - (mean / std) @ w1
    return w1f, b1f[None], w2, b2[None], w3, b3[None]


def _row_biases(ps):
    out = []
    for w, b in ps:
        out.append(w)
        out.append(b[None])
    return out


def _widx(v):
    return jnp.pad(v.astype(jnp.int32), (0, EP - E)).reshape(
        NW, NCHTOT, CHUNK)


def kernel(mesh_position, mesh_properties, mesh_kinematic, obj_position,
           obj_properties, obj_kinematic, om_index, om_attr, mo_index,
           mo_attr, params):
    p = params
    m_x = jnp.concatenate([
        mesh_position[1] - mesh_position[0],
        mesh_position[2] - mesh_position[1],
        mesh_properties, mesh_kinematic], axis=-1)
    o_x = jnp.concatenate([
        obj_position[1] - obj_position[0],
        obj_position[2] - obj_position[1],
        obj_properties, obj_kinematic], axis=-1)

    def _pw(sp):
        om_w1 = sp['om_e'][0][0]
        mo_w1 = sp['mo_e'][0][0]
        return ((om_w1[LAT:2 * LAT], mo_w1[2 * LAT:]),
                (om_w1[2 * LAT:], mo_w1[LAT:2 * LAT]))

    pw0 = _pw(p['steps'][0])
    ns_o, t_om_src, t_mo_dst = _encode_proj(*(
        (o_x,) + _fold_first_layer(p['obj_enc'], p['node_mean'],
                                   p['node_std']) + pw0[0]))
    ns_m, t_om_dst, t_mo_src = _encode_proj(*(
        (m_x,) + _fold_first_layer(p['mesh_enc'], p['node_mean'],
                                   p['node_std']) + pw0[1]))

    pad2 = ((0, EP - E), (0, 0))
    ea_om = jnp.pad(om_attr, pad2)
    ea_mo = jnp.pad(mo_attr, pad2)
    ew_om = _fold_first_layer(p['om_edge_enc'], p['om_mean'], p['om_std'])
    ew_mo = _fold_first_layer(p['mo_edge_enc'], p['mo_mean'], p['mo_std'])

    gidx_om = jnp.stack([_widx(om_index[0]), _widx(om_index[1])], axis=1)
    gidx_mo = jnp.stack([_widx(mo_index[0]), _widx(mo_index[1])], axis=1)
    sidx_om = _widx(om_index[1])
    sidx_mo = _widx(mo_index[1])
    zeros = jnp.zeros((NM, LAT), jnp.float32)
    sc_gather, sc_scatter = _sc_kernels()

    for st in range(STEPS):
        sp = p['steps'][st]
        om_w1, om_b1 = sp['om_e'][0]
        mo_w1, mo_b1 = sp['mo_e'][0]

        g_om_s, g_om_d = sc_gather(t_om_src, t_om_dst, gidx_om)
        g_mo_s, g_mo_d = sc_gather(t_mo_src, t_mo_dst, gidx_mo)

        om_tailw = (om_w1[:LAT], om_b1[None], *_row_biases(sp['om_e'][1:]))
        mo_tailw = (mo_w1[:LAT], mo_b1[None], *_row_biases(sp['mo_e'][1:]))
        if st == 0:
            e_om = _tail0(ea_om, g_om_s, g_om_d, ew_om, *om_tailw)
            e_mo = _tail0(ea_mo, g_mo_s, g_mo_d, ew_mo, *mo_tailw)
        else:
            e_om = _tail(e_om, g_om_s, g_om_d, *om_tailw)
            e_mo = _tail(e_mo, g_mo_s, g_mo_d, *mo_tailw)

        agg_m2 = sc_scatter(e_om, sidx_om, zeros).reshape(2, NM, LAT)
        agg_o2 = sc_scatter(e_mo, sidx_mo, zeros).reshape(2, NM, LAT)

        me_w1, me_b1 = sp['mesh_n'][0]
        ob_w1, ob_b1 = sp['obj_n'][0]
        m_args = (agg_m2, me_w1[:LAT], me_w1[LAT:], me_b1[None],
                  *_row_biases(sp['mesh_n'][1:]))
        o_args = (agg_o2, ob_w1[:LAT], ob_w1[LAT:], ob_b1[None],
                  *_row_biases(sp['obj_n'][1:]))
        if st < STEPS - 1:
            pwn = _pw(p['steps'][st + 1])
            ns_m, t_om_dst, t_mo_src = _nupd_proj(ns_m, *m_args, *pwn[1])
            ns_o, t_om_src, t_mo_dst = _nupd_proj(ns_o, *o_args, *pwn[0])
        else:
            m_acc = _nupd_dec(ns_m, *m_args, _row_biases(p['mesh_dec']))
            o_acc = _nupd_dec(ns_o, *o_args, _row_biases(p['obj_dec']))

    return m_acc, o_acc
